# Initial kernel scaffold; baseline (speedup 1.0000x reference)
import jax
import jax.numpy as jnp
from jax.experimental import pallas as pl


def kernel(x, in_coords, out_coords, weight, bias):
    raise NotImplementedError("write your pallas kernel here")


# speedup vs baseline: 2.0734x; 2.0734x over previous
import functools

import jax
import jax.numpy as jnp
from jax import lax
from jax.experimental import pallas as pl
from jax.experimental.pallas import tpu as pltpu
from jax.experimental.pallas import tpu_sc as plsc

_K = 16
_QT = 128
_RT = 256
_CHUNK = 128


def _knn_body(inT_ref, outc_ref, idx_ref):
    n_in = inT_ref.shape[1]
    ix = inT_ref[0:1, :]
    iy = inT_ref[1:2, :]
    ox = outc_ref[:, 0:1]
    oy = outc_ref[:, 1:2]
    in_sq = ix * ix + iy * iy
    out_sq = ox * ox + oy * oy
    bfc = lambda v: v.astype(jnp.bfloat16).astype(jnp.float32)
    m = bfc(ox) * bfc(ix) + bfc(oy) * bfc(iy)
    d = (in_sq + out_sq) - 2.0 * m
    d = jnp.maximum(d, jnp.float32(1e-12))
    iota = lax.broadcasted_iota(jnp.int32, (d.shape[0], n_in), 1)
    big = jnp.float32(1e30)
    cols = []
    for _ in range(_K):
        m = jnp.min(d, axis=1, keepdims=True)
        hit = d == m
        idx = jnp.min(jnp.where(hit, iota, n_in), axis=1, keepdims=True)
        cols.append(idx)
        d = jnp.where(iota == idx, big, d)
    idx_ref[...] = jnp.concatenate(cols, axis=1)


def _knn_topk(in_coords, out_coords):
    n_in = in_coords.shape[0]
    n_out = out_coords.shape[0]
    inT = in_coords.T
    return pl.pallas_call(
        _knn_body,
        grid=(n_out // _QT,),
        in_specs=[
            pl.BlockSpec((2, n_in), lambda q: (0, 0)),
            pl.BlockSpec((_QT, 2), lambda q: (q, 0)),
        ],
        out_specs=pl.BlockSpec((_QT, _K), lambda q: (q, 0)),
        out_shape=jax.ShapeDtypeStruct((n_out, _K), jnp.int32),
    )(inT, out_coords)


def _make_gather(n_rows_total, rows_per_batch, n_in, c_in):
    info = plsc.get_sparse_core_info()
    nw = info.num_cores * info.num_subcores
    per_w = n_rows_total // nw
    n_chunk = per_w // _CHUNK
    w_per_batch = rows_per_batch // per_w
    mesh = plsc.VectorSubcoreMesh(core_axis_name="c", subcore_axis_name="s")

    @functools.partial(
        pl.kernel,
        mesh=mesh,
        out_type=jax.ShapeDtypeStruct((n_rows_total, c_in), jnp.float32),
        scratch_types=[
            pltpu.VMEM((_CHUNK,), jnp.int32),
            pltpu.VMEM((_CHUNK, c_in), jnp.float32),
            pltpu.SemaphoreType.DMA,
        ],
    )
    def gather_k(x_hbm, idx_hbm, out_hbm, idx_v, rows_v, sem):
        wid = lax.axis_index("s") * info.num_cores + lax.axis_index("c")
        b = wid // w_per_batch
        row0 = wid * per_w

        def body(c, carry):
            base = row0 + c * _CHUNK
            src = base - b * rows_per_batch
            pltpu.sync_copy(idx_hbm.at[pl.ds(src, _CHUNK)], idx_v)
            off = (b * n_in).astype(jnp.int32)
            for i in range(_CHUNK // 16):
                sl = pl.ds(i * 16, 16)
                idx_v[sl] = idx_v[sl] + off
            pltpu.async_copy(x_hbm.at[idx_v], rows_v, sem).wait()
            pltpu.sync_copy(rows_v, out_hbm.at[pl.ds(base, _CHUNK)])
            return carry

        lax.fori_loop(0, n_chunk, body, 0)

    return gather_k


def _mm_body(f_ref, w_ref, b_ref, o_ref):
    acc = lax.dot_general(
        f_ref[...], w_ref[...],
        (((1,), (1,)), ((), ())),
        preferred_element_type=jnp.float32,
    )
    o_ref[...] = acc + b_ref[...]


def _knn_matmul(feats, weight, bias):
    bn, kc = feats.shape
    c_out = weight.shape[0]
    return pl.pallas_call(
        _mm_body,
        grid=(bn // _RT,),
        in_specs=[
            pl.BlockSpec((_RT, kc), lambda r: (r, 0)),
            pl.BlockSpec((c_out, kc), lambda r: (0, 0)),
            pl.BlockSpec((1, c_out), lambda r: (0, 0)),
        ],
        out_specs=pl.BlockSpec((_RT, c_out), lambda r: (r, 0)),
        out_shape=jax.ShapeDtypeStruct((bn, c_out), jnp.float32),
    )(feats, weight, bias.reshape(1, c_out))


def kernel(x, in_coords, out_coords, weight, bias):
    B, n_in, c_in = x.shape
    n_out = out_coords.shape[0]
    c_out = weight.shape[0]

    knn_idx = _knn_topk(in_coords, out_coords)

    x_flat = x.reshape(B * n_in, c_in)
    idx_flat = knn_idx.reshape(n_out * _K)
    n_rows = B * n_out * _K
    gather_fn = _make_gather(n_rows, n_out * _K, n_in, c_in)
    feats = gather_fn(x_flat, idx_flat)

    feats2 = feats.reshape(B * n_out, _K * c_in)
    out = _knn_matmul(feats2, weight, bias)
    return out.reshape(B, n_out, c_out)
